# Initial kernel scaffold; baseline (speedup 1.0000x reference)
import jax
import jax.numpy as jnp
from jax.experimental import pallas as pl


def kernel(x, batch, v, lin):
    raise NotImplementedError("write your pallas kernel here")


# speedup vs baseline: 20.3327x; 20.3327x over previous
import jax
import jax.numpy as jnp
from jax.experimental import pallas as pl

_N = 100000
_F = 128
_T = 32
_S = 32
_G = 128
_NB = 1000
_GRID = _N // _NB


def _body(x_ref, b_ref, v_ref, lin_ref, o_ref):
    i = pl.program_id(0)
    nh50 = jnp.dot(x_ref[...], v_ref[...],
                   preferred_element_type=jnp.float32) * 50.0
    cols = []
    for s in range(_S):
        z = lin_ref[0, s] * 50.0 - nh50
        cols.append(jax.nn.sigmoid(z))
    e = jnp.concatenate(cols, axis=1)
    gid = jax.lax.broadcasted_iota(jnp.int32, (_G, _NB), 0)
    oh = (gid == b_ref[0]).astype(jnp.float32)
    acc = jnp.dot(oh, e, preferred_element_type=jnp.float32)

    @pl.when(i == 0)
    def _():
        o_ref[...] = jnp.zeros_like(o_ref)

    o_ref[...] += acc


@jax.jit
def kernel(x, batch, v, lin):
    b3 = batch.reshape(_GRID, 1, _NB)
    lin2 = lin.reshape(1, _S)
    out2 = pl.pallas_call(
        _body,
        grid=(_GRID,),
        in_specs=[
            pl.BlockSpec((_NB, _F), lambda i: (i, 0)),
            pl.BlockSpec((1, 1, _NB), lambda i: (i, 0, 0)),
            pl.BlockSpec((_F, _T), lambda i: (0, 0)),
            pl.BlockSpec((1, _S), lambda i: (0, 0)),
        ],
        out_specs=pl.BlockSpec((_G, _S * _T), lambda i: (0, 0)),
        out_shape=jax.ShapeDtypeStruct((_G, _S * _T), jnp.float32),
    )(x, b3, v, lin2)
    return out2.reshape(_G, _S, _T)


# speedup vs baseline: 21.2403x; 1.0446x over previous
import jax
import jax.numpy as jnp
from jax.experimental import pallas as pl

_N = 100000
_F = 128
_T = 32
_S = 32
_G = 128
_NB = 2000
_GRID = _N // _NB


def _body(x_ref, b_ref, v_ref, lin_ref, o_ref):
    i = pl.program_id(0)
    nh50 = jnp.dot(x_ref[...], v_ref[...],
                   preferred_element_type=jnp.float32) * 50.0
    cols = []
    for s in range(_S):
        z = lin_ref[0, s] * 50.0 - nh50
        cols.append(jax.nn.sigmoid(z).astype(jnp.bfloat16))
    e = jnp.concatenate(cols, axis=1)
    gid = jax.lax.broadcasted_iota(jnp.int32, (_G, _NB), 0)
    oh = (gid == b_ref[0]).astype(jnp.bfloat16)
    acc = jnp.dot(oh, e, preferred_element_type=jnp.float32)

    @pl.when(i == 0)
    def _():
        o_ref[...] = jnp.zeros_like(o_ref)

    o_ref[...] += acc


@jax.jit
def kernel(x, batch, v, lin):
    b3 = batch.reshape(_GRID, 1, _NB)
    lin2 = lin.reshape(1, _S)
    out2 = pl.pallas_call(
        _body,
        grid=(_GRID,),
        in_specs=[
            pl.BlockSpec((_NB, _F), lambda i: (i, 0)),
            pl.BlockSpec((1, 1, _NB), lambda i: (i, 0, 0)),
            pl.BlockSpec((_F, _T), lambda i: (0, 0)),
            pl.BlockSpec((1, _S), lambda i: (0, 0)),
        ],
        out_specs=pl.BlockSpec((_G, _S * _T), lambda i: (0, 0)),
        out_shape=jax.ShapeDtypeStruct((_G, _S * _T), jnp.float32),
    )(x, b3, v, lin2)
    return out2.reshape(_G, _S, _T)


# speedup vs baseline: 70.6864x; 3.3279x over previous
import jax
import jax.numpy as jnp
from jax.experimental import pallas as pl

_N = 100000
_F = 128
_T = 32
_S = 32
_G = 128
_NB = 2000
_GRID = _N // _NB


def _body(x_ref, b_ref, v_ref, lin_ref, o_ref):
    i = pl.program_id(0)
    nh25t = jax.lax.dot_general(
        v_ref[...], x_ref[...],
        (((0,), (1,)), ((), ())),
        preferred_element_type=jnp.float32) * 25.0
    rows = []
    for s in range(_S):
        z = lin_ref[0, s] * 25.0 - nh25t
        rows.append(jnp.tanh(z).astype(jnp.bfloat16))
    et = jnp.concatenate(rows, axis=0)
    gid = jax.lax.broadcasted_iota(jnp.int32, (_NB, _G), 1)
    oht = (gid == b_ref[0]).astype(jnp.bfloat16)
    acc = jnp.dot(et, oht, preferred_element_type=jnp.float32)
    cnt = jnp.dot(jnp.ones((8, _NB), jnp.bfloat16), oht,
                  preferred_element_type=jnp.float32)[:1]
    acc = 0.5 * acc + 0.5 * cnt

    @pl.when(i == 0)
    def _():
        o_ref[...] = jnp.zeros_like(o_ref)

    o_ref[...] += acc


@jax.jit
def kernel(x, batch, v, lin):
    b3 = batch.reshape(_GRID, _NB, 1)
    lin2 = lin.reshape(1, _S)
    out2 = pl.pallas_call(
        _body,
        grid=(_GRID,),
        in_specs=[
            pl.BlockSpec((_NB, _F), lambda i: (i, 0)),
            pl.BlockSpec((1, _NB, 1), lambda i: (i, 0, 0)),
            pl.BlockSpec((_F, _T), lambda i: (0, 0)),
            pl.BlockSpec((1, _S), lambda i: (0, 0)),
        ],
        out_specs=pl.BlockSpec((_S * _T, _G), lambda i: (0, 0)),
        out_shape=jax.ShapeDtypeStruct((_S * _T, _G), jnp.float32),
    )(x, b3, v, lin2)
    return out2.T.reshape(_G, _S, _T)


# speedup vs baseline: 75.1121x; 1.0626x over previous
import jax
import jax.numpy as jnp
from jax.experimental import pallas as pl

_N = 100000
_F = 128
_T = 32
_S = 32
_G = 128
_NB = 4000
_GRID = _N // _NB


def _body(x_ref, b_ref, v_ref, lin_ref, o_ref):
    i = pl.program_id(0)
    nh25t = jax.lax.dot_general(
        v_ref[...], x_ref[...],
        (((0,), (1,)), ((), ())),
        preferred_element_type=jnp.float32) * 25.0
    rows = []
    for s in range(_S):
        z = lin_ref[0, s] * 25.0 - nh25t
        rows.append(jnp.tanh(z).astype(jnp.bfloat16))
    et = jnp.concatenate(rows, axis=0)
    gid = jax.lax.broadcasted_iota(jnp.int32, (_NB, _G), 1)
    oht = (gid == b_ref[0]).astype(jnp.bfloat16)
    acc = jnp.dot(et, oht, preferred_element_type=jnp.float32)
    cnt = jnp.dot(jnp.ones((8, _NB), jnp.bfloat16), oht,
                  preferred_element_type=jnp.float32)[:1]
    acc = 0.5 * acc + 0.5 * cnt

    @pl.when(i == 0)
    def _():
        o_ref[...] = jnp.zeros_like(o_ref)

    o_ref[...] += acc


@jax.jit
def kernel(x, batch, v, lin):
    b3 = batch.reshape(_GRID, _NB, 1)
    lin2 = lin.reshape(1, _S)
    out2 = pl.pallas_call(
        _body,
        grid=(_GRID,),
        in_specs=[
            pl.BlockSpec((_NB, _F), lambda i: (i, 0)),
            pl.BlockSpec((1, _NB, 1), lambda i: (i, 0, 0)),
            pl.BlockSpec((_F, _T), lambda i: (0, 0)),
            pl.BlockSpec((1, _S), lambda i: (0, 0)),
        ],
        out_specs=pl.BlockSpec((_S * _T, _G), lambda i: (0, 0)),
        out_shape=jax.ShapeDtypeStruct((_S * _T, _G), jnp.float32),
    )(x, b3, v, lin2)
    return out2.T.reshape(_G, _S, _T)


# speedup vs baseline: 116.0419x; 1.5449x over previous
import jax
import jax.numpy as jnp
from jax.experimental import pallas as pl

_N = 100000
_F = 128
_T = 32
_S = 32
_G = 128
_NB = 4000
_GRID = _N // _NB


def _body(x_ref, b_ref, v_ref, lin_ref, o_ref):
    i = pl.program_id(0)
    nh25t = jax.lax.dot_general(
        v_ref[...], x_ref[...],
        (((0,), (1,)), ((), ())),
        preferred_element_type=jnp.float32) * 25.0
    rows = []
    for s in range(_S):
        z = lin_ref[0, s] * 25.0 - nh25t
        rows.append(jnp.tanh(z).astype(jnp.bfloat16))
    et = jnp.concatenate(rows, axis=0)
    gid = jax.lax.broadcasted_iota(jnp.int32, (_G, _NB), 0)
    oh = (gid == b_ref[0]).astype(jnp.bfloat16)
    acc = jax.lax.dot_general(
        et, oh, (((1,), (1,)), ((), ())),
        preferred_element_type=jnp.float32)
    cnt = jax.lax.dot_general(
        jnp.ones((8, _NB), jnp.bfloat16), oh, (((1,), (1,)), ((), ())),
        preferred_element_type=jnp.float32)[:1]
    acc = 0.5 * acc + 0.5 * cnt

    @pl.when(i == 0)
    def _():
        o_ref[...] = jnp.zeros_like(o_ref)

    o_ref[...] += acc


@jax.jit
def kernel(x, batch, v, lin):
    b3 = batch.reshape(_GRID, 1, _NB)
    lin2 = lin.reshape(1, _S)
    out2 = pl.pallas_call(
        _body,
        grid=(_GRID,),
        in_specs=[
            pl.BlockSpec((_NB, _F), lambda i: (i, 0)),
            pl.BlockSpec((1, 1, _NB), lambda i: (i, 0, 0)),
            pl.BlockSpec((_F, _T), lambda i: (0, 0)),
            pl.BlockSpec((1, _S), lambda i: (0, 0)),
        ],
        out_specs=pl.BlockSpec((_S * _T, _G), lambda i: (0, 0)),
        out_shape=jax.ShapeDtypeStruct((_S * _T, _G), jnp.float32),
    )(x, b3, v, lin2)
    return out2.T.reshape(_G, _S, _T)
